# Initial kernel scaffold; baseline (speedup 1.0000x reference)
import jax
import jax.numpy as jnp
from jax.experimental import pallas as pl


def kernel(x):
    raise NotImplementedError("write your pallas kernel here")


# speedup vs baseline: 1.6950x; 1.6950x over previous
import functools

import jax
import jax.numpy as jnp
import numpy as np
from jax import lax
from jax.experimental import pallas as pl
from jax.experimental.pallas import tpu as pltpu
from jax.experimental.pallas import tpu_sc as plsc

_MAX_FRAMES = 12
_TOPK = 64
_N_TOKENS = 197
_D = 512
_B = 16
_FRAMES = _B * _MAX_FRAMES
_OUT_TOK = _TOPK + 1
_NW = 32
_FPW = _FRAMES // _NW
_PAD = 72

_PERM = np.asarray(jax.random.permutation(jax.random.key(42), _N_TOKENS - 1))
_SEL = np.sort(_PERM[:_TOPK])
_FRAME_ROWS = np.concatenate([[0], 1 + _SEL]).astype(np.int32)


def _build_index_table() -> np.ndarray:
    tbl = np.zeros((_NW, _FPW, _PAD), np.int32)
    for w in range(_NW):
        for j in range(_FPW):
            g = w * _FPW + j
            base = g * _N_TOKENS
            tbl[w, j, :_OUT_TOK] = base + _FRAME_ROWS
            tbl[w, j, _OUT_TOK:] = base
    return tbl


_IDX_TBL = _build_index_table()


@functools.partial(
    pl.kernel,
    out_type=jax.ShapeDtypeStruct((_FRAMES * _OUT_TOK, _D), jnp.float32),
    mesh=plsc.VectorSubcoreMesh(core_axis_name="c", subcore_axis_name="s"),
    compiler_params=pltpu.CompilerParams(use_tc_tiling_on_sc=False),
    scratch_types=[
        pltpu.VMEM((_FPW, _PAD), jnp.int32),
        pltpu.VMEM((_PAD, _D), jnp.float32),
        pltpu.VMEM((_PAD, _D), jnp.float32),
        pltpu.SemaphoreType.DMA,
        pltpu.SemaphoreType.DMA,
    ],
)
def _gather_tokens(x_hbm, idx_hbm, out_hbm, idx_v, buf0, buf1, g_sem, w_sem):
    wid = lax.axis_index("s") * 2 + lax.axis_index("c")
    pltpu.sync_copy(idx_hbm.at[wid], idx_v)
    bufs = (buf0, buf1)
    sems = (g_sem, w_sem)
    for j in range(_FPW):
        buf = bufs[j % 2]
        pltpu.async_copy(x_hbm.at[idx_v.at[j]], buf, sems[j % 2]).wait()
        out_base = wid * (_FPW * _OUT_TOK) + j * _OUT_TOK
        pltpu.sync_copy(buf.at[pl.ds(0, _OUT_TOK)],
                        out_hbm.at[pl.ds(out_base, _OUT_TOK)])


def kernel(x):
    B, L, D = x.shape
    x2d = x.reshape(B * L, D)
    out2d = _gather_tokens(x2d, jnp.asarray(_IDX_TBL))
    return out2d.reshape(B, _MAX_FRAMES * _OUT_TOK, D)


# speedup vs baseline: 1.7336x; 1.0228x over previous
import functools

import jax
import jax.numpy as jnp
import numpy as np
from jax import lax
from jax.experimental import pallas as pl
from jax.experimental.pallas import tpu as pltpu
from jax.experimental.pallas import tpu_sc as plsc

_MAX_FRAMES = 12
_TOPK = 64
_N_TOKENS = 197
_D = 512
_B = 16
_FRAMES = _B * _MAX_FRAMES
_OUT_TOK = _TOPK + 1
_NW = 32
_FPW = _FRAMES // _NW
_PAD = 72

_PERM = np.asarray(jax.random.permutation(jax.random.key(42), _N_TOKENS - 1))
_SEL = np.sort(_PERM[:_TOPK])
_FRAME_ROWS = np.concatenate([[0], 1 + _SEL]).astype(np.int32)


def _build_index_table() -> np.ndarray:
    tbl = np.zeros((_NW, _FPW, _PAD), np.int32)
    for w in range(_NW):
        for j in range(_FPW):
            g = w * _FPW + j
            base = g * _N_TOKENS
            tbl[w, j, :_OUT_TOK] = base + _FRAME_ROWS
            tbl[w, j, _OUT_TOK:] = base
    return tbl


_IDX_TBL = _build_index_table()


@functools.partial(
    pl.kernel,
    out_type=jax.ShapeDtypeStruct((_FRAMES * _OUT_TOK, _D), jnp.float32),
    mesh=plsc.VectorSubcoreMesh(core_axis_name="c", subcore_axis_name="s"),
    compiler_params=pltpu.CompilerParams(use_tc_tiling_on_sc=False),
    scratch_types=[
        pltpu.VMEM((_FPW, _PAD), jnp.int32),
        pltpu.VMEM((_PAD, _D), jnp.float32),
        pltpu.VMEM((_PAD, _D), jnp.float32),
        pltpu.VMEM((_PAD, _D), jnp.float32),
        pltpu.SemaphoreType.DMA,
        pltpu.SemaphoreType.DMA,
        pltpu.SemaphoreType.DMA,
        pltpu.SemaphoreType.DMA,
        pltpu.SemaphoreType.DMA,
        pltpu.SemaphoreType.DMA,
    ],
)
def _gather_tokens(x_hbm, idx_hbm, out_hbm, idx_v,
                   buf0, buf1, buf2, gs0, gs1, gs2, ws0, ws1, ws2):
    wid = lax.axis_index("s") * 2 + lax.axis_index("c")
    pltpu.sync_copy(idx_hbm.at[wid], idx_v)
    bufs = (buf0, buf1, buf2)
    gsems = (gs0, gs1, gs2)
    wsems = (ws0, ws1, ws2)
    nbuf = len(bufs)

    def start_gather(j):
        return pltpu.async_copy(x_hbm.at[idx_v.at[j]], bufs[j % nbuf],
                                gsems[j % nbuf])

    def start_store(j):
        out_base = wid * (_FPW * _OUT_TOK) + j * _OUT_TOK
        return pltpu.async_copy(bufs[j % nbuf].at[pl.ds(0, _OUT_TOK)],
                                out_hbm.at[pl.ds(out_base, _OUT_TOK)],
                                wsems[j % nbuf])

    gathers = [start_gather(j) for j in range(nbuf)]
    writes = [None] * _FPW
    for j in range(_FPW):
        gathers[j % nbuf].wait()
        writes[j] = start_store(j)
        nxt = j + nbuf
        if nxt < _FPW:
            writes[j].wait()
            gathers[nxt % nbuf] = start_gather(nxt)
    for j in range(_FPW - nbuf, _FPW):
        writes[j].wait()


def kernel(x):
    B, L, D = x.shape
    x2d = x.reshape(B * L, D)
    out2d = _gather_tokens(x2d, jnp.asarray(_IDX_TBL))
    return out2d.reshape(B, _MAX_FRAMES * _OUT_TOK, D)


# speedup vs baseline: 1.7346x; 1.0006x over previous
import functools

import jax
import jax.numpy as jnp
import numpy as np
from jax import lax
from jax.experimental import pallas as pl
from jax.experimental.pallas import tpu as pltpu
from jax.experimental.pallas import tpu_sc as plsc

_MAX_FRAMES = 12
_TOPK = 64
_N_TOKENS = 197
_D = 512
_B = 16
_L = _MAX_FRAMES * _N_TOKENS
_OUT_TOK = _TOPK + 1
_OUT_L = _MAX_FRAMES * _OUT_TOK
_NW = 32
_FPW = 6
_PAD = 72

_PERM = np.asarray(jax.random.permutation(jax.random.key(42), _N_TOKENS - 1))
_SEL = np.sort(_PERM[:_TOPK])
_FRAME_ROWS = np.concatenate([[0], 1 + _SEL]).astype(np.int32)


def _build_index_table() -> np.ndarray:
    tbl = np.zeros((_NW, _FPW, _PAD), np.int32)
    for w in range(_NW):
        half = w % 2
        for j in range(_FPW):
            f = half * _FPW + j
            base = f * _N_TOKENS
            tbl[w, j, :_OUT_TOK] = base + _FRAME_ROWS
            tbl[w, j, _OUT_TOK:] = base
    return tbl


_IDX_TBL = _build_index_table()


@functools.partial(
    pl.kernel,
    out_type=jax.ShapeDtypeStruct((_B, _OUT_L, _D), jnp.float32),
    mesh=plsc.VectorSubcoreMesh(core_axis_name="c", subcore_axis_name="s"),
    compiler_params=pltpu.CompilerParams(use_tc_tiling_on_sc=False),
    scratch_types=[
        pltpu.VMEM((_FPW, _PAD), jnp.int32),
        pltpu.VMEM((_PAD, _D), jnp.float32),
        pltpu.VMEM((_PAD, _D), jnp.float32),
        pltpu.VMEM((_PAD, _D), jnp.float32),
        pltpu.SemaphoreType.DMA,
        pltpu.SemaphoreType.DMA,
        pltpu.SemaphoreType.DMA,
        pltpu.SemaphoreType.DMA,
        pltpu.SemaphoreType.DMA,
        pltpu.SemaphoreType.DMA,
    ],
)
def _gather_tokens(x_hbm, idx_hbm, out_hbm, idx_v,
                   buf0, buf1, buf2, gs0, gs1, gs2, ws0, ws1, ws2):
    wid = lax.axis_index("s") * 2 + lax.axis_index("c")
    b = wid // 2
    half = wid % 2
    pltpu.sync_copy(idx_hbm.at[wid], idx_v)
    bufs = (buf0, buf1, buf2)
    gsems = (gs0, gs1, gs2)
    wsems = (ws0, ws1, ws2)
    nbuf = len(bufs)

    def start_gather(j):
        return pltpu.async_copy(x_hbm.at[b].at[idx_v.at[j]], bufs[j % nbuf],
                                gsems[j % nbuf])

    def start_store(j):
        out_base = half * (_FPW * _OUT_TOK) + j * _OUT_TOK
        return pltpu.async_copy(bufs[j % nbuf].at[pl.ds(0, _OUT_TOK)],
                                out_hbm.at[b].at[pl.ds(out_base, _OUT_TOK)],
                                wsems[j % nbuf])

    gathers = [start_gather(j) for j in range(nbuf)]
    writes = [None] * _FPW
    for j in range(_FPW):
        gathers[j % nbuf].wait()
        writes[j] = start_store(j)
        nxt = j + nbuf
        if nxt < _FPW:
            writes[j].wait()
            gathers[nxt % nbuf] = start_gather(nxt)
    for j in range(_FPW - nbuf, _FPW):
        writes[j].wait()


def kernel(x):
    return _gather_tokens(x, jnp.asarray(_IDX_TBL))


# speedup vs baseline: 2.7227x; 1.5696x over previous
import functools

import jax
import jax.numpy as jnp
import numpy as np
from jax import lax
from jax.experimental import pallas as pl
from jax.experimental.pallas import tpu as pltpu
from jax.experimental.pallas import tpu_sc as plsc

_MAX_FRAMES = 12
_TOPK = 64
_N_TOKENS = 197
_D = 512
_B = 16
_OUT_TOK = _TOPK + 1
_OUT_L = _MAX_FRAMES * _OUT_TOK
_NW = 32
_NCHUNK = 4
_GATHER_N = 108

_STARTS = ((0, 96, 192, 288), (384, 480, 576, 672))
_LENS = ((96, 96, 96, 96), (96, 96, 96, 104))
_TAIL_N = 16
_TAIL_S = _OUT_L - _TAIL_N

_PERM = np.asarray(jax.random.permutation(jax.random.key(42), _N_TOKENS - 1))
_SEL = np.sort(_PERM[:_TOPK])
_FRAME_ROWS = np.concatenate([[0], 1 + _SEL]).astype(np.int32)

_ROW_MAP = ((np.arange(_OUT_L) // _OUT_TOK) * _N_TOKENS
            + _FRAME_ROWS[np.arange(_OUT_L) % _OUT_TOK]).astype(np.int32)


def _build_index_table() -> np.ndarray:
    tbl = np.zeros((_NW, 8, 128), np.int32)
    for w in range(_NW):
        half = w % 2
        for c in range(_NCHUNK):
            s, n = _STARTS[half][c], _LENS[half][c]
            tbl[w, c, :n] = _ROW_MAP[s:s + n]
            tbl[w, c, n:] = _ROW_MAP[s]
        tbl[w, 4, :_TAIL_N] = _ROW_MAP[_TAIL_S:]
        tbl[w, 4, _TAIL_N:] = _ROW_MAP[_TAIL_S]
    return tbl


_IDX_TBL = _build_index_table()


@functools.partial(
    pl.kernel,
    out_type=jax.ShapeDtypeStruct((_B, _OUT_L, _D), jnp.float32),
    mesh=plsc.VectorSubcoreMesh(core_axis_name="c", subcore_axis_name="s"),
    compiler_params=pltpu.CompilerParams(use_tc_tiling_on_sc=True),
    scratch_types=[
        pltpu.VMEM((8, 128), jnp.int32),
        pltpu.VMEM((_GATHER_N, _D), jnp.float32),
        pltpu.VMEM((_GATHER_N, _D), jnp.float32),
        pltpu.VMEM((_TAIL_N, _D), jnp.float32),
        pltpu.SemaphoreType.DMA,
        pltpu.SemaphoreType.DMA,
        pltpu.SemaphoreType.DMA,
        pltpu.SemaphoreType.DMA,
    ],
)
def _gather_tokens(x_hbm, idx_hbm, out_hbm, idx_v,
                   buf0, buf1, tailbuf, gs0, gs1, ws0, ws1):
    wid = lax.axis_index("s") * 2 + lax.axis_index("c")
    b = wid // 2
    half = wid % 2
    pltpu.sync_copy(idx_hbm.at[wid], idx_v)
    bufs = (buf0, buf1)
    gsems = (gs0, gs1)
    wsems = (ws0, ws1)

    def start_gather(c):
        return pltpu.async_copy(
            x_hbm.at[b].at[idx_v.at[c, pl.ds(0, _GATHER_N)]],
            bufs[c % 2], gsems[c % 2])

    def start_store96(c):
        s = pl.multiple_of(half * 384 + c * 96, 8)
        return pltpu.async_copy(bufs[c % 2].at[pl.ds(0, 96)],
                                out_hbm.at[b].at[pl.ds(s, 96)],
                                wsems[c % 2])

    gathers = [start_gather(0), start_gather(1)]
    writes = [None, None, None]
    for c in range(3):
        gathers[c % 2].wait()
        writes[c] = start_store96(c)
        if c + 2 < _NCHUNK:
            writes[c].wait()
            gathers[c % 2] = start_gather(c + 2)
    gathers[3 % 2].wait()

    @pl.when(half == 0)
    def _():
        pltpu.sync_copy(bufs[1].at[pl.ds(0, 96)],
                        out_hbm.at[b].at[pl.ds(288, 96)])

    @pl.when(half == 1)
    def _():
        pltpu.sync_copy(bufs[1].at[pl.ds(0, 104)],
                        out_hbm.at[b].at[pl.ds(672, 104)])
        pltpu.async_copy(x_hbm.at[b].at[idx_v.at[4, pl.ds(0, _TAIL_N)]],
                         tailbuf, gs0).wait()
        tail_rows = _TAIL_S + lax.iota(jnp.int32, _TAIL_N)
        pltpu.async_copy(tailbuf, out_hbm.at[b].at[tail_rows], ws0).wait()

    writes[2].wait()


def kernel(x):
    return _gather_tokens(x, jnp.asarray(_IDX_TBL))


# speedup vs baseline: 9.2913x; 3.4126x over previous
import functools

import jax
import jax.numpy as jnp
import numpy as np
from jax import lax
from jax.experimental import pallas as pl
from jax.experimental.pallas import tpu as pltpu
from jax.experimental.pallas import tpu_sc as plsc

_MAX_FRAMES = 12
_TOPK = 64
_N_TOKENS = 197
_D = 512
_B = 16
_OUT_TOK = _TOPK + 1
_OUT_L = _MAX_FRAMES * _OUT_TOK
_L = _MAX_FRAMES * _N_TOKENS
_NW = 32
_NCHUNK = 4
_CW = 7
_NCH_TOT = _NW * _NCHUNK
_MAX_START = _OUT_L - _CW

_SEL = np.array([
    2, 3, 4, 5, 7, 16, 19, 29, 30, 31, 34, 35, 37, 39, 42, 44, 45, 56,
    58, 61, 63, 65, 67, 70, 72, 78, 82, 83, 85, 90, 99, 101, 102, 108,
    110, 111, 112, 114, 117, 121, 123, 129, 130, 139, 142, 144, 148, 152,
    153, 155, 156, 157, 163, 167, 174, 175, 176, 177, 178, 179, 183, 186,
    188, 189], dtype=np.int32)
_FRAME_ROWS = np.concatenate([[0], 1 + _SEL]).astype(np.int32)

_ROW_MAP = ((np.arange(_OUT_L) // _OUT_TOK) * _N_TOKENS
            + _FRAME_ROWS[np.arange(_OUT_L) % _OUT_TOK]).astype(np.int32)


def _chunk_start(i: int) -> int:
    return min(i * _OUT_L // _NCH_TOT, _MAX_START)


def _build_index_table() -> np.ndarray:
    tbl = np.zeros((_NW, 8, 128), np.int32)
    for w in range(_NW):
        for c in range(_NCHUNK):
            s = _chunk_start(w * _NCHUNK + c)
            tbl[w, c, :_CW] = _ROW_MAP[s:s + _CW]
    return tbl


_IDX_TBL = _build_index_table()


@functools.partial(
    pl.kernel,
    out_type=jax.ShapeDtypeStruct((_OUT_L, _B, _D), jnp.float32),
    mesh=plsc.VectorSubcoreMesh(core_axis_name="c", subcore_axis_name="s"),
    compiler_params=pltpu.CompilerParams(use_tc_tiling_on_sc=True),
    scratch_types=[
        pltpu.VMEM((8, 128), jnp.int32),
        pltpu.VMEM((_CW, _B, _D), jnp.float32),
        pltpu.VMEM((_CW, _B, _D), jnp.float32),
        pltpu.SemaphoreType.DMA,
        pltpu.SemaphoreType.DMA,
        pltpu.SemaphoreType.DMA,
        pltpu.SemaphoreType.DMA,
    ],
)
def _gather_tokens(x_hbm, idx_hbm, out_hbm, idx_v,
                   buf0, buf1, gs0, gs1, ws0, ws1):
    wid = lax.axis_index("s") * 2 + lax.axis_index("c")
    pltpu.sync_copy(idx_hbm.at[wid], idx_v)
    bufs = (buf0, buf1)
    gsems = (gs0, gs1)
    wsems = (ws0, ws1)

    def start(c):
        i = wid * _NCHUNK + c
        return lax.min(i * _OUT_L // _NCH_TOT, _MAX_START)

    def start_gather(c):
        return pltpu.async_copy(x_hbm.at[idx_v.at[c, pl.ds(0, _CW)]],
                                bufs[c % 2], gsems[c % 2])

    def start_store(c):
        return pltpu.async_copy(bufs[c % 2],
                                out_hbm.at[pl.ds(start(c), _CW)],
                                wsems[c % 2])

    gathers = [start_gather(0), start_gather(1)]
    writes = [None] * _NCHUNK
    for c in range(_NCHUNK):
        gathers[c % 2].wait()
        writes[c] = start_store(c)
        if c + 2 < _NCHUNK:
            writes[c].wait()
            gathers[c % 2] = start_gather(c + 2)
    writes[_NCHUNK - 2].wait()
    writes[_NCHUNK - 1].wait()


def kernel(x):
    xt = jnp.transpose(x, (1, 0, 2))
    out_t = _gather_tokens(xt, jnp.asarray(_IDX_TBL))
    return jnp.transpose(out_t, (1, 0, 2))


# speedup vs baseline: 9.8125x; 1.0561x over previous
import functools

import jax
import jax.numpy as jnp
import numpy as np
from jax import lax
from jax.experimental import pallas as pl
from jax.experimental.pallas import tpu as pltpu
from jax.experimental.pallas import tpu_sc as plsc

_MAX_FRAMES = 12
_TOPK = 64
_N_TOKENS = 197
_D = 512
_B = 16
_OUT_TOK = _TOPK + 1
_OUT_L = _MAX_FRAMES * _OUT_TOK
_L = _MAX_FRAMES * _N_TOKENS
_NW = 32
_NCHUNK = 4
_CW = 7
_C_OFF = (0, 7, 13, 19)
_C_LEN = (7, 6, 6, 6)
_SPAN = 25
_MAX_BASE = _OUT_L - _SPAN

_SEL = np.array([
    2, 3, 4, 5, 7, 16, 19, 29, 30, 31, 34, 35, 37, 39, 42, 44, 45, 56,
    58, 61, 63, 65, 67, 70, 72, 78, 82, 83, 85, 90, 99, 101, 102, 108,
    110, 111, 112, 114, 117, 121, 123, 129, 130, 139, 142, 144, 148, 152,
    153, 155, 156, 157, 163, 167, 174, 175, 176, 177, 178, 179, 183, 186,
    188, 189], dtype=np.int32)
_FRAME_ROWS = np.concatenate([[0], 1 + _SEL]).astype(np.int32)

_ROW_MAP = ((np.arange(_OUT_L) // _OUT_TOK) * _N_TOKENS
            + _FRAME_ROWS[np.arange(_OUT_L) % _OUT_TOK]).astype(np.int32)


def _build_index_table() -> np.ndarray:
    tbl = np.zeros((_NW, 8, 128), np.int32)
    for w in range(_NW):
        base = min(w * _OUT_L // _NW, _MAX_BASE)
        for c in range(_NCHUNK):
            s = base + _C_OFF[c]
            n = _C_LEN[c]
            tbl[w, c, :n] = _ROW_MAP[s:s + n]
    return tbl


_IDX_TBL = _build_index_table()


@functools.partial(
    pl.kernel,
    out_type=jax.ShapeDtypeStruct((_OUT_L, _B, _D), jnp.float32),
    mesh=plsc.VectorSubcoreMesh(core_axis_name="c", subcore_axis_name="s"),
    compiler_params=pltpu.CompilerParams(use_tc_tiling_on_sc=True),
    scratch_types=[
        pltpu.VMEM((8, 128), jnp.int32),
        pltpu.VMEM((_CW, _B, _D), jnp.float32),
        pltpu.VMEM((_CW, _B, _D), jnp.float32),
        pltpu.SemaphoreType.DMA,
        pltpu.SemaphoreType.DMA,
        pltpu.SemaphoreType.DMA,
        pltpu.SemaphoreType.DMA,
    ],
)
def _gather_tokens(x_hbm, idx_hbm, out_hbm, idx_v,
                   buf0, buf1, gs0, gs1, ws0, ws1):
    wid = lax.axis_index("s") * 2 + lax.axis_index("c")
    pltpu.sync_copy(idx_hbm.at[wid], idx_v)
    bufs = (buf0, buf1)
    gsems = (gs0, gs1)
    wsems = (ws0, ws1)

    base = lax.min(wid * _OUT_L // _NW, _MAX_BASE)

    def start_gather(c):
        return pltpu.async_copy(x_hbm.at[idx_v.at[c, pl.ds(0, _C_LEN[c])]],
                                bufs[c % 2].at[pl.ds(0, _C_LEN[c])],
                                gsems[c % 2])

    def start_store(c):
        return pltpu.async_copy(bufs[c % 2].at[pl.ds(0, _C_LEN[c])],
                                out_hbm.at[pl.ds(base + _C_OFF[c], _C_LEN[c])],
                                wsems[c % 2])

    gathers = [start_gather(0), start_gather(1)]
    writes = [None] * _NCHUNK
    for c in range(_NCHUNK):
        gathers[c % 2].wait()
        writes[c] = start_store(c)
        if c + 2 < _NCHUNK:
            writes[c].wait()
            gathers[c % 2] = start_gather(c + 2)
    writes[_NCHUNK - 2].wait()
    writes[_NCHUNK - 1].wait()


def kernel(x):
    xt = jnp.transpose(x, (1, 0, 2))
    out_t = _gather_tokens(xt, jnp.asarray(_IDX_TBL))
    return jnp.transpose(out_t, (1, 0, 2))


# speedup vs baseline: 10.0811x; 1.0274x over previous
import functools

import jax
import jax.numpy as jnp
import numpy as np
from jax import lax
from jax.experimental import pallas as pl
from jax.experimental.pallas import tpu as pltpu
from jax.experimental.pallas import tpu_sc as plsc

_MAX_FRAMES = 12
_TOPK = 64
_N_TOKENS = 197
_D = 512
_B = 16
_OUT_TOK = _TOPK + 1
_OUT_L = _MAX_FRAMES * _OUT_TOK
_L = _MAX_FRAMES * _N_TOKENS
_NW = 32
_NCHUNK = 5
_CW = 5
_NBUF = 3
_C_OFF = (0, 5, 10, 15, 20)
_C_LEN = (5, 5, 5, 5, 5)
_SPAN = 25
_MAX_BASE = _OUT_L - _SPAN

_SEL = np.array([
    2, 3, 4, 5, 7, 16, 19, 29, 30, 31, 34, 35, 37, 39, 42, 44, 45, 56,
    58, 61, 63, 65, 67, 70, 72, 78, 82, 83, 85, 90, 99, 101, 102, 108,
    110, 111, 112, 114, 117, 121, 123, 129, 130, 139, 142, 144, 148, 152,
    153, 155, 156, 157, 163, 167, 174, 175, 176, 177, 178, 179, 183, 186,
    188, 189], dtype=np.int32)
_FRAME_ROWS = np.concatenate([[0], 1 + _SEL]).astype(np.int32)

_ROW_MAP = ((np.arange(_OUT_L) // _OUT_TOK) * _N_TOKENS
            + _FRAME_ROWS[np.arange(_OUT_L) % _OUT_TOK]).astype(np.int32)


def _build_index_table() -> np.ndarray:
    tbl = np.zeros((_NW, 8, 128), np.int32)
    for w in range(_NW):
        base = min(w * _OUT_L // _NW, _MAX_BASE)
        for c in range(_NCHUNK):
            s = base + _C_OFF[c]
            n = _C_LEN[c]
            tbl[w, c, :n] = _ROW_MAP[s:s + n]
    return tbl


_IDX_TBL = _build_index_table()


@functools.partial(
    pl.kernel,
    out_type=jax.ShapeDtypeStruct((_OUT_L, _B, _D), jnp.float32),
    mesh=plsc.VectorSubcoreMesh(core_axis_name="c", subcore_axis_name="s"),
    compiler_params=pltpu.CompilerParams(use_tc_tiling_on_sc=True),
    scratch_types=[
        pltpu.VMEM((8, 128), jnp.int32),
        pltpu.VMEM((_CW, _B, _D), jnp.float32),
        pltpu.VMEM((_CW, _B, _D), jnp.float32),
        pltpu.VMEM((_CW, _B, _D), jnp.float32),
        pltpu.SemaphoreType.DMA,
        pltpu.SemaphoreType.DMA,
        pltpu.SemaphoreType.DMA,
        pltpu.SemaphoreType.DMA,
        pltpu.SemaphoreType.DMA,
        pltpu.SemaphoreType.DMA,
    ],
)
def _gather_tokens(x_hbm, idx_hbm, out_hbm, idx_v,
                   buf0, buf1, buf2, gs0, gs1, gs2, ws0, ws1, ws2):
    wid = lax.axis_index("s") * 2 + lax.axis_index("c")
    pltpu.sync_copy(idx_hbm.at[wid], idx_v)
    bufs = (buf0, buf1, buf2)
    gsems = (gs0, gs1, gs2)
    wsems = (ws0, ws1, ws2)

    base = lax.min(wid * _OUT_L // _NW, _MAX_BASE)

    def start_gather(c):
        return pltpu.async_copy(x_hbm.at[idx_v.at[c, pl.ds(0, _C_LEN[c])]],
                                bufs[c % _NBUF], gsems[c % _NBUF])

    def start_store(c):
        return pltpu.async_copy(bufs[c % _NBUF],
                                out_hbm.at[pl.ds(base + _C_OFF[c], _C_LEN[c])],
                                wsems[c % _NBUF])

    gathers = [start_gather(c) for c in range(_NBUF)]
    writes = [None] * _NCHUNK
    waited = set()
    for c in range(_NCHUNK):
        gathers[c % _NBUF].wait()
        writes[c] = start_store(c)
        if c + _NBUF < _NCHUNK:
            writes[c].wait()
            waited.add(c)
            gathers[c % _NBUF] = start_gather(c + _NBUF)
    for c in range(_NCHUNK):
        if c not in waited:
            writes[c].wait()


def kernel(x):
    xt = jnp.transpose(x, (1, 0, 2))
    out_t = _gather_tokens(xt, jnp.asarray(_IDX_TBL))
    return jnp.transpose(out_t, (1, 0, 2))
